# baseline (speedup 1.0000x reference)
import functools

import jax
import jax.numpy as jnp
from jax import lax
from jax.experimental import pallas as pl
from jax.experimental.pallas import tpu as pltpu
from jax.experimental.pallas import tpu_sc as plsc

R = 128
N = 32768
K = 512
L = 16
NBIN = 256
CH = N // L
SSTRIDE = 8
SCH = CH // SSTRIDE
SAMPLE_MIN = 150
INT_MIN = -2147483648


def _keys(bs):
    negs = [b < 0 for b in bs]
    alts = [jnp.int32(INT_MIN) - b for b in bs]
    return [jnp.where(n, a, b) for n, a, b in zip(negs, alts, bs)]


def _build():
    info = plsc.get_sparse_core_info()
    nc, ns = info.num_cores, info.num_subcores
    nw = nc * ns
    rows_per_w = R // nw
    mesh = plsc.VectorSubcoreMesh(core_axis_name="c", subcore_axis_name="s")

    @functools.partial(
        pl.kernel,
        mesh=mesh,
        out_type=jax.ShapeDtypeStruct((R, K), jnp.float32),
        compiler_params=pltpu.CompilerParams(needs_layout_passes=False),
        scratch_types=[
            pltpu.VMEM((2 * N,), jnp.float32),
            pltpu.VMEM((N + 4 * L,), jnp.float32),
            pltpu.VMEM((NBIN * L,), jnp.int32),
            pltpu.VMEM((L * L,), jnp.int32),
            pltpu.VMEM((NBIN,), jnp.int32),
            pltpu.VMEM((K + L,), jnp.float32),
            pltpu.SemaphoreType.DMA,
        ],
    )
    def kmax(x_hbm, o_hbm, row2_v, cand_v, hist_v, hist4_v, tot_v, out_v,
             dsem):
        wid = lax.axis_index("s") * nc + lax.axis_index("c")
        iota = lax.iota(jnp.int32, L)
        iota16 = lax.shift_left(iota, 4)
        ones = jnp.ones((L,), jnp.int32)
        zeros = jnp.zeros((L,), jnp.int32)

        def clear_hist(i, c):
            hist_v[pl.ds(i * L, L)] = zeros
            return c

        lax.fori_loop(0, NBIN, clear_hist, 0)

        def clear_hist4(i, c):
            hist4_v[pl.ds(i * L, L)] = zeros
            return c

        lax.fori_loop(0, L, clear_hist4, 0)

        def bins8(ks):
            hs = [lax.shift_right_arithmetic(k, 24) for k in ks]
            hs = [lax.bitwise_and(h, jnp.int32(255)) for h in hs]
            hs = [lax.bitwise_xor(h, jnp.int32(128)) for h in hs]
            return [lax.shift_left(h, 4) + iota for h in hs]

        def totals(g, c):
            base_addr = lax.shift_left(g, 8)
            acc = plsc.load_gather(hist_v, [base_addr + iota16])
            for l in range(1, L):
                acc = acc + plsc.load_gather(hist_v, [base_addr + iota16 + l])
            tot_v[pl.ds(lax.shift_left(g, 4), L)] = acc
            for u in range(L):
                hist_v[pl.ds(base_addr + u * L, L)] = zeros
            return c

        def find_bin(i, carry):
            carry_in = carry
            g = L - 1 - i
            tv = tot_v[pl.ds(lax.shift_left(g, 4), L)]
            for u in range(L):
                rem, bsel, found = carry_in
                lane = L - 1 - u
                b = lax.shift_left(g, 4) + lane
                cnt = tv[lane]
                take = (found == 0) & (cnt >= rem)
                carry_in = (
                    jnp.where((found == 0) & (cnt < rem), rem - cnt, rem),
                    jnp.where(take, b, bsel),
                    jnp.where(take, jnp.int32(1), found))
            return carry_in

        pltpu.async_copy(x_hbm.at[wid * rows_per_w],
                         row2_v.at[pl.ds(0, N)], dsem)

        def do_row(j, c):
            row = wid * rows_per_w + j
            sbase = pl.multiple_of(
                lax.shift_left(lax.bitwise_and(j, 1), 15), N)
            pltpu.make_async_copy(x_hbm.at[row],
                                  row2_v.at[pl.ds(sbase, N)], dsem).wait()

            @pl.when(j < rows_per_w - 1)
            def _():
                nbase = pl.multiple_of(
                    lax.shift_left(lax.bitwise_and(j + 1, 1), 15), N)
                pltpu.async_copy(x_hbm.at[row + 1],
                                 row2_v.at[pl.ds(nbase, N)], dsem)

            def sscan(i, c):
                base = lax.shift_left(i, 2)
                vs = [row2_v[pl.ds(sbase + (base + u) * (L * SSTRIDE), L)]
                      for u in range(4)]
                bs = [lax.bitcast_convert_type(v, jnp.int32) for v in vs]
                idxs = bins8(_keys(bs))
                for u in range(4):
                    plsc.addupdate_scatter(hist_v, [idxs[u]], ones)
                return c

            lax.fori_loop(0, SCH // 4, sscan, 0)
            lax.fori_loop(0, L, totals, 0)
            rems, b0s, _ = lax.fori_loop(
                0, L, find_bin,
                (jnp.int32(SAMPLE_MIN), jnp.int32(0), jnp.int32(0)))
            pv8 = lax.bitwise_xor(b0s, jnp.int32(128))

            def sscan2(i, c):
                base = lax.shift_left(i, 2)
                vs = [row2_v[pl.ds(sbase + (base + u) * (L * SSTRIDE), L)]
                      for u in range(4)]
                bs = [lax.bitcast_convert_type(v, jnp.int32) for v in vs]
                ks = _keys(bs)
                hs = [lax.bitwise_and(
                    lax.shift_right_arithmetic(k, 24), jnp.int32(255))
                    for k in ks]
                masks = [h == pv8 for h in hs]
                sb = [lax.bitwise_and(
                    lax.shift_right_arithmetic(k, 20), jnp.int32(15))
                    for k in ks]
                idxs = [lax.shift_left(b, 4) + iota for b in sb]
                for u in range(4):
                    plsc.addupdate_scatter(hist4_v, [idxs[u]], ones,
                                           mask=masks[u])
                return c

            lax.fori_loop(0, SCH // 4, sscan2, 0)
            acc4 = plsc.load_gather(hist4_v, [iota16])
            for l in range(1, L):
                acc4 = acc4 + plsc.load_gather(hist4_v, [iota16 + l])
            for u in range(L):
                hist4_v[pl.ds(u * L, L)] = zeros
            carrys = (rems, jnp.int32(0), jnp.int32(0))
            for u in range(L):
                remc, bsel, found = carrys
                lane = L - 1 - u
                cnt = acc4[lane]
                take = (found == 0) & (cnt >= remc)
                carrys = (jnp.where((found == 0) & (cnt < remc),
                                    remc - cnt, remc),
                          jnp.where(take, jnp.int32(lane), bsel),
                          jnp.where(take, jnp.int32(1), found))
            _, sub4, _ = carrys
            t_lo = lax.shift_left(
                lax.bitwise_or(lax.shift_left(pv8, 4), sub4), 20)
            floor_bits = jnp.where(
                t_lo == INT_MIN,
                jnp.int32(0xFF800000 - (1 << 32)),
                jnp.where(t_lo >= 0, t_lo, INT_MIN - t_lo))
            floor_f = lax.bitcast_convert_type(
                jnp.broadcast_to(floor_bits, (L,)), jnp.float32)

            def compact_cand(i, ptr):
                base = lax.shift_left(i, 5)
                vs = [row2_v[pl.ds(sbase + (base + u) * L, L)]
                      for u in range(32)]
                sels = [v >= floor_f for v in vs]
                pcs = [plsc.all_reduce_population_count(s)[0] for s in sels]
                for u in range(32):
                    plsc.store_compressed(cand_v.at[pl.ds(ptr, L)], vs[u],
                                          mask=sels[u])
                    ptr = ptr + pcs[u]
                return ptr

            ncand = lax.fori_loop(0, CH // 32, compact_cand, jnp.int32(0))

            @pl.when(ncand < K)
            def _():
                def copy_all(i, c):
                    cand_v[pl.ds(i * L, L)] = row2_v[pl.ds(sbase + i * L, L)]
                    return c
                lax.fori_loop(0, CH, copy_all, 0)

            ncand = jnp.where(ncand < K, jnp.int32(N), ncand)
            ncc4 = lax.div(ncand + (4 * L - 1), jnp.int32(4 * L))

            def cscan8(i, c):
                base = lax.shift_left(i, 2)
                vs = [cand_v[pl.ds((base + u) * L, L)] for u in range(4)]
                bs = [lax.bitcast_convert_type(v, jnp.int32) for v in vs]
                idxs = bins8(_keys(bs))
                inbs = [(lax.shift_left(base + u, 4) + iota) < ncand
                        for u in range(4)]
                for u in range(4):
                    plsc.addupdate_scatter(hist_v, [idxs[u]], ones,
                                           mask=inbs[u])
                return c

            lax.fori_loop(0, ncc4, cscan8, 0)
            lax.fori_loop(0, L, totals, 0)
            rem, b0, _ = lax.fori_loop(
                0, L, find_bin, (jnp.int32(K), jnp.int32(0), jnp.int32(0)))
            pv = lax.bitwise_xor(b0, jnp.int32(128))

            def refine(rem, pv, rnd):
                msh = 24 - 4 * (rnd - 1)
                mmask = (1 << (8 + 4 * (rnd - 1))) - 1
                bsh = 24 - 4 * rnd

                def scan(i, c):
                    base = lax.shift_left(i, 2)
                    vs = [cand_v[pl.ds((base + u) * L, L)] for u in range(4)]
                    bs = [lax.bitcast_convert_type(v, jnp.int32) for v in vs]
                    ks = _keys(bs)
                    mvs = [lax.bitwise_and(
                        lax.shift_right_arithmetic(k, msh), jnp.int32(mmask))
                        for k in ks]
                    inbs = [(lax.shift_left(base + u, 4) + iota) < ncand
                            for u in range(4)]
                    masks = [(mv == pv) & inb for mv, inb in zip(mvs, inbs)]
                    bsv = [lax.bitwise_and(
                        lax.shift_right_arithmetic(k, bsh), jnp.int32(15))
                        for k in ks]
                    idxs = [lax.shift_left(b, 4) + iota for b in bsv]
                    for u in range(4):
                        plsc.addupdate_scatter(hist4_v, [idxs[u]], ones,
                                               mask=masks[u])
                    return c

                lax.fori_loop(0, ncc4, scan, 0)

                acc = plsc.load_gather(hist4_v, [iota16])
                for l in range(1, L):
                    acc = acc + plsc.load_gather(hist4_v, [iota16 + l])
                for u in range(L):
                    hist4_v[pl.ds(u * L, L)] = zeros

                carry4 = (rem, jnp.int32(0), jnp.int32(0))
                for u in range(L):
                    remc, bsel, found = carry4
                    lane = L - 1 - u
                    cnt = acc[lane]
                    take = (found == 0) & (cnt >= remc)
                    carry4 = (jnp.where((found == 0) & (cnt < remc),
                                        remc - cnt, remc),
                              jnp.where(take, jnp.int32(lane), bsel),
                              jnp.where(take, jnp.int32(1), found))
                rem2, b2, _ = carry4
                return rem2, lax.bitwise_or(lax.shift_left(pv, 4), b2)

            for rnd in range(1, 7):
                rem, pv = refine(rem, pv, rnd)

            t = pv
            m = rem

            def emit(i, carry):
                ptr, tiec = carry
                base = lax.shift_left(i, 2)
                vs = [cand_v[pl.ds((base + u) * L, L)] for u in range(4)]
                bs = [lax.bitcast_convert_type(v, jnp.int32) for v in vs]
                ks = _keys(bs)
                inbs = [(lax.shift_left(base + u, 4) + iota) < ncand
                        for u in range(4)]
                gts = [(k > t) & inb for k, inb in zip(ks, inbs)]
                eqs = [(k == t) & inb for k, inb in zip(ks, inbs)]
                eqis = [jnp.where(eq, jnp.int32(1), jnp.int32(0))
                        for eq in eqs]
                excs = [plsc.cumsum(eqi) - eqi for eqi in eqis]
                pceqs = [plsc.all_reduce_population_count(eq)[0]
                         for eq in eqs]
                for u in range(4):
                    sel = gts[u] | (eqs[u] & ((excs[u] + tiec) < m))
                    plsc.store_compressed(out_v.at[pl.ds(ptr, L)], vs[u],
                                          mask=sel)
                    ptr = ptr + plsc.all_reduce_population_count(sel)[0]
                    tiec = tiec + pceqs[u]
                return (ptr, tiec)

            lax.fori_loop(0, ncc4, emit, (jnp.int32(0), jnp.int32(0)))
            pltpu.sync_copy(out_v.at[pl.ds(0, K)], o_hbm.at[row])
            return c

        lax.fori_loop(0, rows_per_w, do_row, 0)

    return kmax


_kmax = _build()


def kernel(x, dim):
    del dim
    return _kmax(x)


# speedup vs baseline: 1.0744x; 1.0744x over previous
import functools

import jax
import jax.numpy as jnp
from jax import lax
from jax.experimental import pallas as pl
from jax.experimental.pallas import tpu as pltpu
from jax.experimental.pallas import tpu_sc as plsc

R = 128
N = 32768
K = 512
L = 16
NBIN = 256
CH = N // L
SSTRIDE = 8
SCH = CH // SSTRIDE
SAMPLE_MIN = 150
CAP = 8192
INT_MIN = -2147483648


def _keys(bs):
    negs = [b < 0 for b in bs]
    alts = [jnp.int32(INT_MIN) - b for b in bs]
    return [jnp.where(n, a, b) for n, a, b in zip(negs, alts, bs)]


def _build():
    info = plsc.get_sparse_core_info()
    nc, ns = info.num_cores, info.num_subcores
    nw = nc * ns
    rows_per_w = R // nw
    mesh = plsc.VectorSubcoreMesh(core_axis_name="c", subcore_axis_name="s")

    @functools.partial(
        pl.kernel,
        mesh=mesh,
        out_type=jax.ShapeDtypeStruct((R, K), jnp.float32),
        compiler_params=pltpu.CompilerParams(needs_layout_passes=False),
        scratch_types=[
            pltpu.VMEM((2 * N,), jnp.float32),
            pltpu.VMEM((N + 4 * L,), jnp.float32),
            pltpu.VMEM((NBIN * L,), jnp.int32),
            pltpu.VMEM((L * L,), jnp.int32),
            pltpu.VMEM((NBIN,), jnp.int32),
            pltpu.VMEM((K + L,), jnp.float32),
            pltpu.VMEM((CAP + 4 * L,), jnp.float32),
            pltpu.SemaphoreType.DMA,
        ],
    )
    def kmax(x_hbm, o_hbm, row2_v, cand_v, hist_v, hist4_v, tot_v, out_v,
             cand2_v, dsem):
        wid = lax.axis_index("s") * nc + lax.axis_index("c")
        iota = lax.iota(jnp.int32, L)
        iota16 = lax.shift_left(iota, 4)
        ones = jnp.ones((L,), jnp.int32)
        zeros = jnp.zeros((L,), jnp.int32)

        def clear_hist(i, c):
            hist_v[pl.ds(i * L, L)] = zeros
            return c

        lax.fori_loop(0, NBIN, clear_hist, 0)

        def clear_hist4(i, c):
            hist4_v[pl.ds(i * L, L)] = zeros
            return c

        lax.fori_loop(0, L, clear_hist4, 0)

        def bins8(ks):
            hs = [lax.shift_right_arithmetic(k, 24) for k in ks]
            hs = [lax.bitwise_and(h, jnp.int32(255)) for h in hs]
            hs = [lax.bitwise_xor(h, jnp.int32(128)) for h in hs]
            return [lax.shift_left(h, 4) + iota for h in hs]

        def totals(g, c):
            base_addr = lax.shift_left(g, 8)
            acc = plsc.load_gather(hist_v, [base_addr + iota16])
            for l in range(1, L):
                acc = acc + plsc.load_gather(hist_v, [base_addr + iota16 + l])
            tot_v[pl.ds(lax.shift_left(g, 4), L)] = acc
            for u in range(L):
                hist_v[pl.ds(base_addr + u * L, L)] = zeros
            return c

        def find_bin(i, carry):
            carry_in = carry
            g = L - 1 - i
            tv = tot_v[pl.ds(lax.shift_left(g, 4), L)]
            for u in range(L):
                rem, bsel, found = carry_in
                lane = L - 1 - u
                b = lax.shift_left(g, 4) + lane
                cnt = tv[lane]
                take = (found == 0) & (cnt >= rem)
                carry_in = (
                    jnp.where((found == 0) & (cnt < rem), rem - cnt, rem),
                    jnp.where(take, b, bsel),
                    jnp.where(take, jnp.int32(1), found))
            return carry_in

        pltpu.async_copy(x_hbm.at[wid * rows_per_w],
                         row2_v.at[pl.ds(0, N)], dsem)

        def do_row(j, c):
            row = wid * rows_per_w + j
            sbase = pl.multiple_of(
                lax.shift_left(lax.bitwise_and(j, 1), 15), N)
            pltpu.make_async_copy(x_hbm.at[row],
                                  row2_v.at[pl.ds(sbase, N)], dsem).wait()

            @pl.when(j < rows_per_w - 1)
            def _():
                nbase = pl.multiple_of(
                    lax.shift_left(lax.bitwise_and(j + 1, 1), 15), N)
                pltpu.async_copy(x_hbm.at[row + 1],
                                 row2_v.at[pl.ds(nbase, N)], dsem)

            def sscan(i, c):
                base = lax.shift_left(i, 2)
                vs = [row2_v[pl.ds(sbase + (base + u) * (L * SSTRIDE), L)]
                      for u in range(4)]
                bs = [lax.bitcast_convert_type(v, jnp.int32) for v in vs]
                idxs = bins8(_keys(bs))
                for u in range(4):
                    plsc.addupdate_scatter(hist_v, [idxs[u]], ones)
                return c

            lax.fori_loop(0, SCH // 4, sscan, 0)
            lax.fori_loop(0, L, totals, 0)
            rems, b0s, _ = lax.fori_loop(
                0, L, find_bin,
                (jnp.int32(SAMPLE_MIN), jnp.int32(0), jnp.int32(0)))
            pv8 = lax.bitwise_xor(b0s, jnp.int32(128))

            def sscan2(i, c):
                base = lax.shift_left(i, 2)
                vs = [row2_v[pl.ds(sbase + (base + u) * (L * SSTRIDE), L)]
                      for u in range(4)]
                bs = [lax.bitcast_convert_type(v, jnp.int32) for v in vs]
                ks = _keys(bs)
                hs = [lax.bitwise_and(
                    lax.shift_right_arithmetic(k, 24), jnp.int32(255))
                    for k in ks]
                masks = [h == pv8 for h in hs]
                sb = [lax.bitwise_and(
                    lax.shift_right_arithmetic(k, 20), jnp.int32(15))
                    for k in ks]
                idxs = [lax.shift_left(b, 4) + iota for b in sb]
                for u in range(4):
                    plsc.addupdate_scatter(hist4_v, [idxs[u]], ones,
                                           mask=masks[u])
                return c

            lax.fori_loop(0, SCH // 4, sscan2, 0)
            acc4 = plsc.load_gather(hist4_v, [iota16])
            for l in range(1, L):
                acc4 = acc4 + plsc.load_gather(hist4_v, [iota16 + l])
            for u in range(L):
                hist4_v[pl.ds(u * L, L)] = zeros
            carrys = (rems, jnp.int32(0), jnp.int32(0))
            for u in range(L):
                remc, bsel, found = carrys
                lane = L - 1 - u
                cnt = acc4[lane]
                take = (found == 0) & (cnt >= remc)
                carrys = (jnp.where((found == 0) & (cnt < remc),
                                    remc - cnt, remc),
                          jnp.where(take, jnp.int32(lane), bsel),
                          jnp.where(take, jnp.int32(1), found))
            _, sub4, _ = carrys
            t_lo = lax.shift_left(
                lax.bitwise_or(lax.shift_left(pv8, 4), sub4), 20)
            floor_bits = jnp.where(
                t_lo == INT_MIN,
                jnp.int32(0xFF800000 - (1 << 32)),
                jnp.where(t_lo >= 0, t_lo, INT_MIN - t_lo))
            floor_f = lax.bitcast_convert_type(
                jnp.broadcast_to(floor_bits, (L,)), jnp.float32)

            def compact_cand(i, ptr):
                base = lax.shift_left(i, 4)
                vs = [row2_v[pl.ds(sbase + (base + u) * L, L)]
                      for u in range(16)]
                sels = [v >= floor_f for v in vs]
                pcs = [plsc.all_reduce_population_count(s)[0] for s in sels]
                for u in range(16):
                    plsc.store_compressed(cand_v.at[pl.ds(ptr, L)], vs[u],
                                          mask=sels[u])
                    ptr = ptr + pcs[u]
                return ptr

            ncand = lax.fori_loop(0, CH // 16, compact_cand, jnp.int32(0))

            @pl.when(ncand < K)
            def _():
                def copy_all(i, c):
                    cand_v[pl.ds(i * L, L)] = row2_v[pl.ds(sbase + i * L, L)]
                    return c
                lax.fori_loop(0, CH, copy_all, 0)

            ncand = jnp.where(ncand < K, jnp.int32(N), ncand)
            ncc4 = lax.div(ncand + (4 * L - 1), jnp.int32(4 * L))

            def cscan8(i, c):
                base = lax.shift_left(i, 2)
                vs = [cand_v[pl.ds((base + u) * L, L)] for u in range(4)]
                bs = [lax.bitcast_convert_type(v, jnp.int32) for v in vs]
                idxs = bins8(_keys(bs))
                inbs = [(lax.shift_left(base + u, 4) + iota) < ncand
                        for u in range(4)]
                for u in range(4):
                    plsc.addupdate_scatter(hist_v, [idxs[u]], ones,
                                           mask=inbs[u])
                return c

            lax.fori_loop(0, ncc4, cscan8, 0)
            lax.fori_loop(0, L, totals, 0)
            rem, b0, _ = lax.fori_loop(
                0, L, find_bin, (jnp.int32(K), jnp.int32(0), jnp.int32(0)))
            pv = lax.bitwise_xor(b0, jnp.int32(128))

            def refine(src_ref, cnt, rem, pv, rnd):
                msh = 24 - 4 * (rnd - 1)
                mmask = (1 << (8 + 4 * (rnd - 1))) - 1
                bsh = 24 - 4 * rnd
                nloops = lax.div(cnt + (4 * L - 1), jnp.int32(4 * L))

                def scan(i, c):
                    base = lax.shift_left(i, 2)
                    vs = [src_ref[pl.ds((base + u) * L, L)]
                          for u in range(4)]
                    bs = [lax.bitcast_convert_type(v, jnp.int32) for v in vs]
                    ks = _keys(bs)
                    mvs = [lax.bitwise_and(
                        lax.shift_right_arithmetic(k, msh), jnp.int32(mmask))
                        for k in ks]
                    inbs = [(lax.shift_left(base + u, 4) + iota) < cnt
                            for u in range(4)]
                    masks = [(mv == pv) & inb for mv, inb in zip(mvs, inbs)]
                    bsv = [lax.bitwise_and(
                        lax.shift_right_arithmetic(k, bsh), jnp.int32(15))
                        for k in ks]
                    idxs = [lax.shift_left(b, 4) + iota for b in bsv]
                    for u in range(4):
                        plsc.addupdate_scatter(hist4_v, [idxs[u]], ones,
                                               mask=masks[u])
                    return c

                lax.fori_loop(0, nloops, scan, 0)

                acc = plsc.load_gather(hist4_v, [iota16])
                for l in range(1, L):
                    acc = acc + plsc.load_gather(hist4_v, [iota16 + l])
                for u in range(L):
                    hist4_v[pl.ds(u * L, L)] = zeros

                carry4 = (rem, jnp.int32(0), jnp.int32(0), jnp.int32(0))
                for u in range(L):
                    remc, bsel, found, csel = carry4
                    lane = L - 1 - u
                    cnt4 = acc[lane]
                    take = (found == 0) & (cnt4 >= remc)
                    carry4 = (jnp.where((found == 0) & (cnt4 < remc),
                                        remc - cnt4, remc),
                              jnp.where(take, jnp.int32(lane), bsel),
                              jnp.where(take, jnp.int32(1), found),
                              jnp.where(take, cnt4, csel))
                rem2, b2, _, csel = carry4
                return (rem2, lax.bitwise_or(lax.shift_left(pv, 4), b2),
                        csel)

            rem, pv, c12 = refine(cand_v, ncand, rem, pv, 1)

            def narrow_rounds(_):
                def ncompact(i, ptr):
                    base = lax.shift_left(i, 2)
                    vs = [cand_v[pl.ds((base + u) * L, L)]
                          for u in range(4)]
                    bs = [lax.bitcast_convert_type(v, jnp.int32)
                          for v in vs]
                    ks = _keys(bs)
                    inbs = [(lax.shift_left(base + u, 4) + iota) < ncand
                            for u in range(4)]
                    sels = [(lax.bitwise_and(
                        lax.shift_right_arithmetic(k, 20),
                        jnp.int32(0xFFF)) == pv) & inb
                        for k, inb in zip(ks, inbs)]
                    pcs = [plsc.all_reduce_population_count(s)[0]
                           for s in sels]
                    for u in range(4):
                        plsc.store_compressed(cand2_v.at[pl.ds(ptr, L)],
                                              vs[u], mask=sels[u])
                        ptr = ptr + pcs[u]
                    return ptr

                ncc4n = lax.div(ncand + (4 * L - 1), jnp.int32(4 * L))
                lax.fori_loop(0, ncc4n, ncompact, jnp.int32(0))
                r, p = rem, pv
                for rnd in range(2, 7):
                    r, p, _ = refine(cand2_v, c12, r, p, rnd)
                return (r, p)

            def wide_rounds(_):
                r, p = rem, pv
                for rnd in range(2, 7):
                    r, p, _ = refine(cand_v, ncand, r, p, rnd)
                return (r, p)

            rem, pv = lax.cond(c12 <= CAP, narrow_rounds, wide_rounds, 0)

            t = pv
            m = rem

            def emit(i, carry):
                ptr, tiec = carry
                base = lax.shift_left(i, 2)
                vs = [cand_v[pl.ds((base + u) * L, L)] for u in range(4)]
                bs = [lax.bitcast_convert_type(v, jnp.int32) for v in vs]
                ks = _keys(bs)
                inbs = [(lax.shift_left(base + u, 4) + iota) < ncand
                        for u in range(4)]
                gts = [(k > t) & inb for k, inb in zip(ks, inbs)]
                eqs = [(k == t) & inb for k, inb in zip(ks, inbs)]
                eqis = [jnp.where(eq, jnp.int32(1), jnp.int32(0))
                        for eq in eqs]
                excs = [plsc.cumsum(eqi) - eqi for eqi in eqis]
                pceqs = [plsc.all_reduce_population_count(eq)[0]
                         for eq in eqs]
                for u in range(4):
                    sel = gts[u] | (eqs[u] & ((excs[u] + tiec) < m))
                    plsc.store_compressed(out_v.at[pl.ds(ptr, L)], vs[u],
                                          mask=sel)
                    ptr = ptr + plsc.all_reduce_population_count(sel)[0]
                    tiec = tiec + pceqs[u]
                return (ptr, tiec)

            lax.fori_loop(0, ncc4, emit, (jnp.int32(0), jnp.int32(0)))
            pltpu.sync_copy(out_v.at[pl.ds(0, K)], o_hbm.at[row])
            return c

        lax.fori_loop(0, rows_per_w, do_row, 0)

    return kmax


_kmax = _build()


def kernel(x, dim):
    del dim
    return _kmax(x)


# speedup vs baseline: 1.0919x; 1.0162x over previous
import functools

import jax
import jax.numpy as jnp
from jax import lax
from jax.experimental import pallas as pl
from jax.experimental.pallas import tpu as pltpu
from jax.experimental.pallas import tpu_sc as plsc

R = 128
N = 32768
K = 512
L = 16
NBIN = 256
CH = N // L
SSTRIDE = 8
SCH = CH // SSTRIDE
SAMPLE_MIN = 150
INT_MIN = -2147483648


def _keys(bs):
    negs = [b < 0 for b in bs]
    alts = [jnp.int32(INT_MIN) - b for b in bs]
    return [jnp.where(n, a, b) for n, a, b in zip(negs, alts, bs)]


def _build():
    info = plsc.get_sparse_core_info()
    nc, ns = info.num_cores, info.num_subcores
    nw = nc * ns
    rows_per_w = R // nw
    mesh = plsc.VectorSubcoreMesh(core_axis_name="c", subcore_axis_name="s")

    @functools.partial(
        pl.kernel,
        mesh=mesh,
        out_type=jax.ShapeDtypeStruct((R, K), jnp.float32),
        compiler_params=pltpu.CompilerParams(needs_layout_passes=False),
        scratch_types=[
            pltpu.VMEM((2 * N,), jnp.float32),
            pltpu.VMEM((N + 4 * L,), jnp.float32),
            pltpu.VMEM((NBIN * L,), jnp.int32),
            pltpu.VMEM((L * L,), jnp.int32),
            pltpu.VMEM((NBIN,), jnp.int32),
            pltpu.VMEM((K + L,), jnp.float32),
            pltpu.SemaphoreType.DMA,
        ],
    )
    def kmax(x_hbm, o_hbm, row2_v, cand_v, hist_v, hist4_v, tot_v, out_v,
             dsem):
        wid = lax.axis_index("s") * nc + lax.axis_index("c")
        iota = lax.iota(jnp.int32, L)
        iota16 = lax.shift_left(iota, 4)
        ones = jnp.ones((L,), jnp.int32)
        zeros = jnp.zeros((L,), jnp.int32)

        def clear_hist(i, c):
            hist_v[pl.ds(i * L, L)] = zeros
            return c

        lax.fori_loop(0, NBIN, clear_hist, 0)

        def clear_hist4(i, c):
            hist4_v[pl.ds(i * L, L)] = zeros
            return c

        lax.fori_loop(0, L, clear_hist4, 0)

        def bins8(ks):
            hs = [lax.shift_right_arithmetic(k, 24) for k in ks]
            hs = [lax.bitwise_and(h, jnp.int32(255)) for h in hs]
            hs = [lax.bitwise_xor(h, jnp.int32(128)) for h in hs]
            return [lax.shift_left(h, 4) + iota for h in hs]

        def totals(g, c):
            base_addr = lax.shift_left(g, 8)
            acc = plsc.load_gather(hist_v, [base_addr + iota16])
            for l in range(1, L):
                acc = acc + plsc.load_gather(hist_v, [base_addr + iota16 + l])
            tot_v[pl.ds(lax.shift_left(g, 4), L)] = acc
            for u in range(L):
                hist_v[pl.ds(base_addr + u * L, L)] = zeros
            return c

        def find_bin(i, carry):
            carry_in = carry
            g = L - 1 - i
            tv = tot_v[pl.ds(lax.shift_left(g, 4), L)]
            for u in range(L):
                rem, bsel, found = carry_in
                lane = L - 1 - u
                b = lax.shift_left(g, 4) + lane
                cnt = tv[lane]
                take = (found == 0) & (cnt >= rem)
                carry_in = (
                    jnp.where((found == 0) & (cnt < rem), rem - cnt, rem),
                    jnp.where(take, b, bsel),
                    jnp.where(take, jnp.int32(1), found))
            return carry_in

        pltpu.async_copy(x_hbm.at[wid * rows_per_w],
                         row2_v.at[pl.ds(0, N)], dsem)

        def do_row(j, c):
            row = wid * rows_per_w + j
            sbase = pl.multiple_of(
                lax.shift_left(lax.bitwise_and(j, 1), 15), N)
            pltpu.make_async_copy(x_hbm.at[row],
                                  row2_v.at[pl.ds(sbase, N)], dsem).wait()

            @pl.when(j < rows_per_w - 1)
            def _():
                nbase = pl.multiple_of(
                    lax.shift_left(lax.bitwise_and(j + 1, 1), 15), N)
                pltpu.async_copy(x_hbm.at[row + 1],
                                 row2_v.at[pl.ds(nbase, N)], dsem)

            def sscan(i, c):
                base = lax.shift_left(i, 2)
                vs = [row2_v[pl.ds(sbase + (base + u) * (L * SSTRIDE), L)]
                      for u in range(4)]
                bs = [lax.bitcast_convert_type(v, jnp.int32) for v in vs]
                idxs = bins8(_keys(bs))
                for u in range(4):
                    plsc.addupdate_scatter(hist_v, [idxs[u]], ones)
                return c

            lax.fori_loop(0, SCH // 4, sscan, 0)
            lax.fori_loop(0, L, totals, 0)
            rems, b0s, _ = lax.fori_loop(
                0, L, find_bin,
                (jnp.int32(SAMPLE_MIN), jnp.int32(0), jnp.int32(0)))
            pv8 = lax.bitwise_xor(b0s, jnp.int32(128))

            def sscan2(i, c):
                base = lax.shift_left(i, 2)
                vs = [row2_v[pl.ds(sbase + (base + u) * (L * SSTRIDE), L)]
                      for u in range(4)]
                bs = [lax.bitcast_convert_type(v, jnp.int32) for v in vs]
                ks = _keys(bs)
                hs = [lax.bitwise_and(
                    lax.shift_right_arithmetic(k, 24), jnp.int32(255))
                    for k in ks]
                masks = [h == pv8 for h in hs]
                sb = [lax.bitwise_and(
                    lax.shift_right_arithmetic(k, 20), jnp.int32(15))
                    for k in ks]
                idxs = [lax.shift_left(b, 4) + iota for b in sb]
                for u in range(4):
                    plsc.addupdate_scatter(hist4_v, [idxs[u]], ones,
                                           mask=masks[u])
                return c

            lax.fori_loop(0, SCH // 4, sscan2, 0)
            acc4 = plsc.load_gather(hist4_v, [iota16])
            for l in range(1, L):
                acc4 = acc4 + plsc.load_gather(hist4_v, [iota16 + l])
            for u in range(L):
                hist4_v[pl.ds(u * L, L)] = zeros
            carrys = (rems, jnp.int32(0), jnp.int32(0))
            for u in range(L):
                remc, bsel, found = carrys
                lane = L - 1 - u
                cnt = acc4[lane]
                take = (found == 0) & (cnt >= remc)
                carrys = (jnp.where((found == 0) & (cnt < remc),
                                    remc - cnt, remc),
                          jnp.where(take, jnp.int32(lane), bsel),
                          jnp.where(take, jnp.int32(1), found))
            _, sub4, _ = carrys
            t_lo = lax.shift_left(
                lax.bitwise_or(lax.shift_left(pv8, 4), sub4), 20)
            floor_bits = jnp.where(
                t_lo == INT_MIN,
                jnp.int32(0xFF800000 - (1 << 32)),
                jnp.where(t_lo >= 0, t_lo, INT_MIN - t_lo))
            floor_f = lax.bitcast_convert_type(
                jnp.broadcast_to(floor_bits, (L,)), jnp.float32)

            def compact_cand(i, ptr):
                base = lax.shift_left(i, 4)
                vs = [row2_v[pl.ds(sbase + (base + u) * L, L)]
                      for u in range(16)]
                sels = [v >= floor_f for v in vs]
                pcs = [plsc.all_reduce_population_count(s)[0] for s in sels]
                for u in range(16):
                    plsc.store_compressed(cand_v.at[pl.ds(ptr, L)], vs[u],
                                          mask=sels[u])
                    ptr = ptr + pcs[u]
                return ptr

            ncand = lax.fori_loop(0, CH // 16, compact_cand, jnp.int32(0))

            @pl.when(ncand < K)
            def _():
                def copy_all(i, c):
                    cand_v[pl.ds(i * L, L)] = row2_v[pl.ds(sbase + i * L, L)]
                    return c
                lax.fori_loop(0, CH, copy_all, 0)

            ncand = jnp.where(ncand < K, jnp.int32(N), ncand)
            ncc4 = lax.div(ncand + (4 * L - 1), jnp.int32(4 * L))

            def cscan8(i, c):
                base = lax.shift_left(i, 2)
                vs = [cand_v[pl.ds((base + u) * L, L)] for u in range(4)]
                bs = [lax.bitcast_convert_type(v, jnp.int32) for v in vs]
                idxs = bins8(_keys(bs))
                inbs = [(lax.shift_left(base + u, 4) + iota) < ncand
                        for u in range(4)]
                for u in range(4):
                    plsc.addupdate_scatter(hist_v, [idxs[u]], ones,
                                           mask=inbs[u])
                return c

            lax.fori_loop(0, ncc4, cscan8, 0)
            lax.fori_loop(0, L, totals, 0)
            rem, b0, _ = lax.fori_loop(
                0, L, find_bin, (jnp.int32(K), jnp.int32(0), jnp.int32(0)))
            pv = lax.bitwise_xor(b0, jnp.int32(128))

            def refine(src_ref, cnt, rem, pv, rnd):
                msh = 24 - 4 * (rnd - 1)
                mmask = (1 << (8 + 4 * (rnd - 1))) - 1
                bsh = 24 - 4 * rnd
                nloops = lax.div(cnt + (4 * L - 1), jnp.int32(4 * L))

                def scan(i, c):
                    base = lax.shift_left(i, 2)
                    vs = [src_ref[pl.ds((base + u) * L, L)]
                          for u in range(4)]
                    bs = [lax.bitcast_convert_type(v, jnp.int32) for v in vs]
                    ks = _keys(bs)
                    mvs = [lax.bitwise_and(
                        lax.shift_right_arithmetic(k, msh), jnp.int32(mmask))
                        for k in ks]
                    inbs = [(lax.shift_left(base + u, 4) + iota) < cnt
                            for u in range(4)]
                    masks = [(mv == pv) & inb for mv, inb in zip(mvs, inbs)]
                    bsv = [lax.bitwise_and(
                        lax.shift_right_arithmetic(k, bsh), jnp.int32(15))
                        for k in ks]
                    idxs = [lax.shift_left(b, 4) + iota for b in bsv]
                    for u in range(4):
                        plsc.addupdate_scatter(hist4_v, [idxs[u]], ones,
                                               mask=masks[u])
                    return c

                lax.fori_loop(0, nloops, scan, 0)

                acc = plsc.load_gather(hist4_v, [iota16])
                for l in range(1, L):
                    acc = acc + plsc.load_gather(hist4_v, [iota16 + l])
                for u in range(L):
                    hist4_v[pl.ds(u * L, L)] = zeros

                carry4 = (rem, jnp.int32(0), jnp.int32(0), jnp.int32(0))
                for u in range(L):
                    remc, bsel, found, csel = carry4
                    lane = L - 1 - u
                    cnt4 = acc[lane]
                    take = (found == 0) & (cnt4 >= remc)
                    carry4 = (jnp.where((found == 0) & (cnt4 < remc),
                                        remc - cnt4, remc),
                              jnp.where(take, jnp.int32(lane), bsel),
                              jnp.where(take, jnp.int32(1), found),
                              jnp.where(take, cnt4, csel))
                rem2, b2, _, csel = carry4
                return (rem2, lax.bitwise_or(lax.shift_left(pv, 4), b2),
                        csel)

            for rnd in range(1, 7):
                rem, pv, _ = refine(cand_v, ncand, rem, pv, rnd)

            t = pv
            m = rem

            def emit(i, carry):
                ptr, tiec = carry
                base = lax.shift_left(i, 2)
                vs = [cand_v[pl.ds((base + u) * L, L)] for u in range(4)]
                bs = [lax.bitcast_convert_type(v, jnp.int32) for v in vs]
                ks = _keys(bs)
                inbs = [(lax.shift_left(base + u, 4) + iota) < ncand
                        for u in range(4)]
                gts = [(k > t) & inb for k, inb in zip(ks, inbs)]
                eqs = [(k == t) & inb for k, inb in zip(ks, inbs)]
                eqis = [jnp.where(eq, jnp.int32(1), jnp.int32(0))
                        for eq in eqs]
                excs = [plsc.cumsum(eqi) - eqi for eqi in eqis]
                pceqs = [plsc.all_reduce_population_count(eq)[0]
                         for eq in eqs]
                for u in range(4):
                    sel = gts[u] | (eqs[u] & ((excs[u] + tiec) < m))
                    plsc.store_compressed(out_v.at[pl.ds(ptr, L)], vs[u],
                                          mask=sel)
                    ptr = ptr + plsc.all_reduce_population_count(sel)[0]
                    tiec = tiec + pceqs[u]
                return (ptr, tiec)

            lax.fori_loop(0, ncc4, emit, (jnp.int32(0), jnp.int32(0)))
            pltpu.sync_copy(out_v.at[pl.ds(0, K)], o_hbm.at[row])
            return c

        lax.fori_loop(0, rows_per_w, do_row, 0)

    return kmax


_kmax = _build()


def kernel(x, dim):
    del dim
    return _kmax(x)
